# baseline (speedup 1.0000x reference)
import jax
import jax.numpy as jnp
from jax.experimental import pallas as pl
from jax.experimental.pallas import tpu as pltpu

_D = 768
_TOP_K = 64
_TS = 2048


def _weff_kernel(pf_ref, perm_ref, w_ref, nw_ref, weff_ref):
    f = jnp.abs(pf_ref[...])
    bits = jax.lax.bitcast_convert_type(f, jnp.int32)

    B = f.shape[0]
    zero = jnp.zeros((B, 1), jnp.int32)

    def body(i, cur):
        cand = cur | (jnp.int32(1) << (jnp.int32(30) - i))
        cnt = jnp.sum((bits >= cand).astype(jnp.int32), axis=1,
                      keepdims=True)
        return jnp.where(cnt >= _TOP_K, cand, cur)

    t = jax.lax.fori_loop(0, 31, body, zero)

    gt = (bits > t).astype(jnp.float32)
    tie = (bits == t).astype(jnp.float32)
    need = jnp.float32(_TOP_K) - jnp.sum(gt, axis=1, keepdims=True)
    i0 = jax.lax.broadcasted_iota(jnp.int32, (_D, _D), 0)
    i1 = jax.lax.broadcasted_iota(jnp.int32, (_D, _D), 1)
    ltm = (i1 < i0).astype(jnp.float32)
    rank = jax.lax.dot_general(
        tie, ltm, (((1,), (1,)), ((), ())),
        preferred_element_type=jnp.float32)
    pre_mask = gt + tie * (rank < need).astype(jnp.float32)

    onehot = (i0 == perm_ref[...]).astype(jnp.float32)
    mask = jax.lax.dot_general(
        pre_mask, onehot, (((1,), (1,)), ((), ())),
        preferred_element_type=jnp.float32)

    w = w_ref[...]
    nw = nw_ref[...]
    for b in range(B):
        weff_ref[b] = w + mask[b:b + 1, :] * nw


def _matmul_kernel(weff_ref, x_ref, out_ref):
    x_tile = x_ref[0]
    out_ref[0] = jax.lax.dot_general(
        x_tile, weff_ref[0], (((1,), (1,)), ((), ())),
        preferred_element_type=jnp.float32)


def kernel(x, W, new_W, perm, prompt_boundary):
    B, S, D = x.shape
    pf = jax.lax.dynamic_index_in_dim(x, prompt_boundary, axis=1,
                                      keepdims=False)
    perm2 = perm.astype(jnp.int32).reshape(1, D)

    weff = pl.pallas_call(
        _weff_kernel,
        out_shape=jax.ShapeDtypeStruct((B, D, D), jnp.float32),
    )(pf, perm2, W, new_W)

    grid = (B, S // _TS)
    return pl.pallas_call(
        _matmul_kernel,
        grid=grid,
        in_specs=[
            pl.BlockSpec((1, D, D), lambda b, s: (b, 0, 0)),
            pl.BlockSpec((1, _TS, D), lambda b, s: (b, s, 0)),
        ],
        out_specs=pl.BlockSpec((1, _TS, D), lambda b, s: (b, s, 0)),
        out_shape=jax.ShapeDtypeStruct((B, S, D), jnp.float32),
        compiler_params=pltpu.CompilerParams(
            dimension_semantics=("parallel", "parallel")),
    )(weff, x)


# speedup vs baseline: 1.2152x; 1.2152x over previous
import jax
import jax.numpy as jnp
from jax.experimental import pallas as pl
from jax.experimental.pallas import tpu as pltpu

_D = 768
_TOP_K = 64
_TS = 2048


def _fused_kernel(pf_ref, perm_ref, w_ref, nw_ref, x_ref, out_ref,
                  mask_ref, weff_ref):
    b = pl.program_id(0)
    s = pl.program_id(1)

    @pl.when((b == 0) & (s == 0))
    def _build_masks():
        f = jnp.abs(pf_ref[...])
        bits = jax.lax.bitcast_convert_type(f, jnp.int32)
        B = f.shape[0]

        def body(i, cur):
            cand = cur | (jnp.int32(1) << (jnp.int32(30) - i))
            cnt = jnp.sum((bits >= cand).astype(jnp.int32), axis=1,
                          keepdims=True)
            return jnp.where(cnt >= _TOP_K, cand, cur)

        t = jax.lax.fori_loop(0, 31, body, jnp.zeros((B, 1), jnp.int32))

        gt = (bits > t).astype(jnp.float32)
        tie = (bits == t).astype(jnp.float32)
        need = jnp.float32(_TOP_K) - jnp.sum(gt, axis=1, keepdims=True)
        i0 = jax.lax.broadcasted_iota(jnp.int32, (_D, _D), 0)
        i1 = jax.lax.broadcasted_iota(jnp.int32, (_D, _D), 1)
        ltm = (i1 < i0).astype(jnp.float32)
        rank = jax.lax.dot_general(
            tie, ltm, (((1,), (1,)), ((), ())),
            preferred_element_type=jnp.float32)
        pre_mask = gt + tie * (rank < need).astype(jnp.float32)

        onehot = (i0 == perm_ref[...]).astype(jnp.float32)
        mask_ref[...] = jax.lax.dot_general(
            pre_mask, onehot, (((1,), (1,)), ((), ())),
            preferred_element_type=jnp.float32)

    @pl.when(s == 0)
    def _build_weff():
        weff_ref[...] = (w_ref[...]
                         + mask_ref[pl.ds(b, 1), :] * nw_ref[...]
                         ).astype(jnp.bfloat16)

    x_tile = x_ref[0].astype(jnp.bfloat16)
    out_ref[0] = jax.lax.dot_general(
        x_tile, weff_ref[...], (((1,), (1,)), ((), ())),
        preferred_element_type=jnp.float32)


def kernel(x, W, new_W, perm, prompt_boundary):
    B, S, D = x.shape
    pf = jax.lax.dynamic_index_in_dim(x, prompt_boundary, axis=1,
                                      keepdims=False)
    perm2 = perm.astype(jnp.int32).reshape(1, D)

    grid = (B, S // _TS)
    return pl.pallas_call(
        _fused_kernel,
        grid=grid,
        in_specs=[
            pl.BlockSpec((B, D), lambda b, s: (0, 0)),
            pl.BlockSpec((1, D), lambda b, s: (0, 0)),
            pl.BlockSpec((D, D), lambda b, s: (0, 0)),
            pl.BlockSpec((D, D), lambda b, s: (0, 0)),
            pl.BlockSpec((1, _TS, D), lambda b, s: (b, s, 0)),
        ],
        out_specs=pl.BlockSpec((1, _TS, D), lambda b, s: (b, s, 0)),
        out_shape=jax.ShapeDtypeStruct((B, S, D), jnp.float32),
        scratch_shapes=[pltpu.VMEM((4, _D), jnp.float32),
                        pltpu.VMEM((_D, _D), jnp.bfloat16)],
        compiler_params=pltpu.CompilerParams(
            dimension_semantics=("arbitrary", "arbitrary")),
    )(pf, perm2, W, new_W, x)
